# Initial kernel scaffold; baseline (speedup 1.0000x reference)
import jax
import jax.numpy as jnp
from jax.experimental import pallas as pl


def kernel(features, prototypes):
    raise NotImplementedError("write your pallas kernel here")


# speedup vs baseline: 2.9343x; 2.9343x over previous
import jax
import jax.numpy as jnp
from jax.experimental import pallas as pl
from jax.experimental.pallas import tpu as pltpu

_BM = 1024
_BN = 2048
_LANES = 128


def _km_kernel(f_ref, p_ref, sim_ref, ids_ref, amax_ref, aidx_ref):
    j = pl.program_id(1)
    nj = pl.num_programs(1)

    f = f_ref[...]
    p = p_ref[...]
    fn = jnp.sqrt(jnp.sum(f * f, axis=1, keepdims=True))
    f = f / jnp.maximum(fn, 1e-12)
    pn = jnp.sqrt(jnp.sum(p * p, axis=1, keepdims=True))
    p = p / jnp.maximum(pn, 1e-12)

    sim = jax.lax.dot_general(f, p, (((1,), (1,)), ((), ())),
                              preferred_element_type=jnp.float32)
    sim_ref[...] = sim

    @pl.when(j == 0)
    def _init():
        amax_ref[...] = jnp.full_like(amax_ref[...], -jnp.inf)
        aidx_ref[...] = jnp.zeros_like(aidx_ref[...])

    bn = sim.shape[1]
    chunks = bn // _LANES
    amax = amax_ref[...]
    aidx = aidx_ref[...]
    for k in range(chunks):
        v = sim[:, k * _LANES:(k + 1) * _LANES]
        chunk_id = j * chunks + k
        gt = v > amax
        amax = jnp.where(gt, v, amax)
        aidx = jnp.where(gt, chunk_id, aidx)
    amax_ref[...] = amax
    aidx_ref[...] = aidx

    @pl.when(j == nj - 1)
    def _finalize():
        a = amax_ref[...]
        ai = aidx_ref[...]
        rowmax = jnp.max(a, axis=1, keepdims=True)
        lane = jax.lax.broadcasted_iota(jnp.int32, a.shape, 1)
        col = ai * _LANES + lane
        cand = jnp.where(a == rowmax, col, jnp.iinfo(jnp.int32).max)
        ids_ref[...] = jnp.min(cand, axis=1, keepdims=True)


def kernel(features, prototypes):
    m, k = features.shape
    n = prototypes.shape[0]
    sim, ids = pl.pallas_call(
        _km_kernel,
        grid=(m // _BM, n // _BN),
        in_specs=[
            pl.BlockSpec((_BM, k), lambda i, j: (i, 0)),
            pl.BlockSpec((_BN, k), lambda i, j: (j, 0)),
        ],
        out_specs=[
            pl.BlockSpec((_BM, _BN), lambda i, j: (i, j)),
            pl.BlockSpec((_BM, 1), lambda i, j: (i, 0)),
        ],
        out_shape=[
            jax.ShapeDtypeStruct((m, n), jnp.float32),
            jax.ShapeDtypeStruct((m, 1), jnp.int32),
        ],
        scratch_shapes=[
            pltpu.VMEM((_BM, _LANES), jnp.float32),
            pltpu.VMEM((_BM, _LANES), jnp.int32),
        ],
        compiler_params=pltpu.CompilerParams(
            dimension_semantics=("parallel", "arbitrary"),
        ),
    )(features, prototypes)
    return ids.reshape(m), sim


# speedup vs baseline: 3.6447x; 1.2421x over previous
import jax
import jax.numpy as jnp
from jax.experimental import pallas as pl
from jax.experimental.pallas import tpu as pltpu

_BM = 2048
_BN = 2048
_SUB = 512
_LANES = 128


def _km_kernel(f_ref, p_ref, sim_ref, ids_ref, amax_ref, aidx_ref):
    j = pl.program_id(1)
    nj = pl.num_programs(1)

    f = f_ref[...]
    p = p_ref[...]
    fn = jnp.sqrt(jnp.sum(f * f, axis=1, keepdims=True))
    f = f / jnp.maximum(fn, 1e-12)
    pn = jnp.sqrt(jnp.sum(p * p, axis=1, keepdims=True))
    p = p / jnp.maximum(pn, 1e-12)

    @pl.when(j == 0)
    def _init():
        amax_ref[...] = jnp.full_like(amax_ref[...], -jnp.inf)
        aidx_ref[...] = jnp.zeros_like(aidx_ref[...])

    bn = sim_ref.shape[1]
    chunks = bn // _LANES
    sub_chunks = _SUB // _LANES
    amax = amax_ref[...]
    aidx = aidx_ref[...]
    for s in range(bn // _SUB):
        ps = p[s * _SUB:(s + 1) * _SUB, :]
        v = jax.lax.dot_general(f, ps, (((1,), (1,)), ((), ())),
                                preferred_element_type=jnp.float32)
        sim_ref[:, s * _SUB:(s + 1) * _SUB] = v
        for k in range(sub_chunks):
            vv = v[:, k * _LANES:(k + 1) * _LANES]
            chunk_id = j * chunks + s * sub_chunks + k
            gt = vv > amax
            amax = jnp.where(gt, vv, amax)
            aidx = jnp.where(gt, chunk_id, aidx)
    amax_ref[...] = amax
    aidx_ref[...] = aidx

    @pl.when(j == nj - 1)
    def _finalize():
        a = amax_ref[...]
        ai = aidx_ref[...]
        rowmax = jnp.max(a, axis=1, keepdims=True)
        lane = jax.lax.broadcasted_iota(jnp.int32, a.shape, 1)
        col = ai * _LANES + lane
        cand = jnp.where(a == rowmax, col, jnp.iinfo(jnp.int32).max)
        ids_ref[...] = jnp.min(cand, axis=1, keepdims=True)


def kernel(features, prototypes):
    m, k = features.shape
    n = prototypes.shape[0]
    sim, ids = pl.pallas_call(
        _km_kernel,
        grid=(m // _BM, n // _BN),
        in_specs=[
            pl.BlockSpec((_BM, k), lambda i, j: (i, 0)),
            pl.BlockSpec((_BN, k), lambda i, j: (j, 0)),
        ],
        out_specs=[
            pl.BlockSpec((_BM, _BN), lambda i, j: (i, j)),
            pl.BlockSpec((_BM, 1), lambda i, j: (i, 0)),
        ],
        out_shape=[
            jax.ShapeDtypeStruct((m, n), jnp.float32),
            jax.ShapeDtypeStruct((m, 1), jnp.int32),
        ],
        scratch_shapes=[
            pltpu.VMEM((_BM, _LANES), jnp.float32),
            pltpu.VMEM((_BM, _LANES), jnp.int32),
        ],
        compiler_params=pltpu.CompilerParams(
            dimension_semantics=("parallel", "arbitrary"),
        ),
    )(features, prototypes)
    return ids.reshape(m), sim
